# baseline (speedup 1.0000x reference)
import functools

import jax
import jax.numpy as jnp
from jax import lax
from jax.experimental import pallas as pl
from jax.experimental.pallas import tpu as pltpu
from jax.experimental.pallas import tpu_sc as plsc

B = 128
V = 100000
NW = 32
RPW = B // NW
LANES = 16
NTRIO = 5
CH = 20000
NCH = V // CH
NVC = CH // (LANES * NTRIO)
NEG = float("-inf")


def _merge_sorted3(a, b, c, a2, b2, c2):
    x1 = jnp.maximum(a, a2)
    y1 = jnp.minimum(a, a2)
    x2 = jnp.maximum(b, b2)
    y2 = jnp.minimum(b, b2)
    x3 = jnp.maximum(c, c2)
    m2 = jnp.maximum(y1, x2)
    m3 = jnp.maximum(jnp.minimum(y1, x2), jnp.maximum(y2, x3))
    return x1, m2, m3


def _make_sc_call():
    mesh = plsc.VectorSubcoreMesh(core_axis_name="c", subcore_axis_name="s")

    @functools.partial(
        pl.kernel,
        mesh=mesh,
        compiler_params=pltpu.CompilerParams(needs_layout_passes=False, use_tc_tiling_on_sc=False),
        out_type=jax.ShapeDtypeStruct((NW, LANES), jnp.float32),
        scratch_types=[
            pltpu.VMEM((CH,), jnp.float32),
            pltpu.VMEM((CH,), jnp.float32),
            pltpu.VMEM((LANES,), jnp.int32),
            pltpu.VMEM((LANES,), jnp.float32),
            pltpu.VMEM((LANES,), jnp.float32),
            pltpu.VMEM((LANES,), jnp.float32),
            pltpu.VMEM((LANES,), jnp.float32),
            pltpu.SemaphoreType.DMA,
            pltpu.SemaphoreType.DMA,
        ],
    )
    def dlr_loss_sc(logits_hbm, ypad_hbm, out_hbm,
                    buf0, buf1, yv, av, bv, cv, outv, sem0, sem1):
        wid = lax.axis_index("s") * 2 + lax.axis_index("c")
        pltpu.sync_copy(ypad_hbm.at[wid], yv)
        yvec = yv[...]
        iota = lax.iota(jnp.int32, LANES)
        bufs = (buf0, buf1)
        sems = (sem0, sem1)

        def start(k):
            r, kc = divmod(k, NCH)
            return pltpu.async_copy(
                logits_hbm.at[wid * RPW + r, pl.ds(kc * CH, CH)],
                bufs[k % 2], sems[k % 2])

        out_acc = jnp.zeros((LANES,), jnp.float32)
        ninf = jnp.full((LANES,), NEG, dtype=jnp.float32)

        handle = start(0)
        trios = (ninf,) * (3 * NTRIO)
        zy_acc = ninf
        for k in range(RPW * NCH):
            r, kc = divmod(k, NCH)
            handle.wait()
            if k + 1 < RPW * NCH:
                handle = start(k + 1)
            buf = bufs[k % 2]

            def body(j, carry, buf=buf):
                new = []
                base = j * (LANES * NTRIO)
                for t in range(NTRIO):
                    a, b, c = carry[3 * t:3 * t + 3]
                    x = buf[pl.ds(base + t * LANES, LANES)]
                    a2 = jnp.maximum(a, x)
                    tt = jnp.minimum(a, x)
                    b2 = jnp.maximum(b, tt)
                    tt2 = jnp.minimum(b, tt)
                    c2 = jnp.maximum(c, tt2)
                    new += [a2, b2, c2]
                return tuple(new)

            trios = lax.fori_loop(0, NVC, body, trios)

            rel = yvec - kc * CH
            in_ch = (rel >= 0) & (rel < CH)
            relc = jnp.clip(rel, 0, CH - 1)
            g = plsc.load_gather(buf, [relc])
            zy_acc = jnp.where(in_ch, g, zy_acc)

            if kc == NCH - 1:
                a, b, c = trios[0:3]
                for t in range(1, NTRIO):
                    a, b, c = _merge_sorted3(a, b, c, *trios[3 * t:3 * t + 3])
                for off in (8, 4, 2, 1):
                    av[...] = a
                    bv[...] = b
                    cv[...] = c
                    ix = jnp.bitwise_xor(iota, off)
                    a_s = plsc.load_gather(av, [ix])
                    b_s = plsc.load_gather(bv, [ix])
                    c_s = plsc.load_gather(cv, [ix])
                    a, b, c = _merge_sorted3(a, b, c, a_s, b_s, c_s)
                z_other = jnp.where((zy_acc == a) & (b < a), b, a)
                scale = a - c + jnp.float32(1e-12)
                loss_vec = -(zy_acc - z_other) / scale
                out_acc = jnp.where(iota == r, loss_vec, out_acc)
                trios = (ninf,) * (3 * NTRIO)
                zy_acc = ninf
        outv[...] = out_acc
        pltpu.sync_copy(outv, out_hbm.at[wid])

    return dlr_loss_sc


_sc_call = _make_sc_call()


def kernel(logits, y_true):
    y32 = y_true.astype(jnp.int32)
    ypad = jnp.zeros((NW, LANES), jnp.int32).at[:, :RPW].set(
        y32.reshape(NW, RPW))
    out = _sc_call(logits, ypad)
    return out[:, :RPW].reshape(B)


# speedup vs baseline: 1.0372x; 1.0372x over previous
import functools

import jax
import jax.numpy as jnp
from jax import lax
from jax.experimental import pallas as pl
from jax.experimental.pallas import tpu as pltpu
from jax.experimental.pallas import tpu_sc as plsc

B = 128
V = 100000
NSUB = 16
RPW = 4
LANES = 16
NTRIO = 5
NVREG = V // (LANES * NTRIO)
NEG = float("-inf")


def _merge_sorted3(a, b, c, a2, b2, c2):
    x1 = jnp.maximum(a, a2)
    y1 = jnp.minimum(a, a2)
    x2 = jnp.maximum(b, b2)
    y2 = jnp.minimum(b, b2)
    x3 = jnp.maximum(c, c2)
    m2 = jnp.maximum(y1, x2)
    m3 = jnp.maximum(jnp.minimum(y1, x2), jnp.maximum(y2, x3))
    return x1, m2, m3


def _make_sc_call(row_base):
    mesh = plsc.VectorSubcoreMesh(core_axis_name="c", subcore_axis_name="s",
                                  num_cores=1)

    @functools.partial(
        pl.kernel,
        mesh=mesh,
        compiler_params=pltpu.CompilerParams(needs_layout_passes=False),
        out_type=jax.ShapeDtypeStruct((NSUB, LANES), jnp.float32),
        scratch_types=[
            pltpu.VMEM((V,), jnp.float32),
            pltpu.VMEM((LANES,), jnp.int32),
            pltpu.VMEM((LANES,), jnp.float32),
            pltpu.VMEM((LANES,), jnp.float32),
            pltpu.VMEM((LANES,), jnp.float32),
            pltpu.VMEM((LANES,), jnp.float32),
        ],
    )
    def dlr_loss_sc(logits_hbm, ypad_hbm, out_hbm, buf, yv, av, bv, cv, outv):
        wid = lax.axis_index("s")
        pltpu.sync_copy(ypad_hbm.at[wid], yv)
        yvec = yv[...]
        iota = lax.iota(jnp.int32, LANES)
        out_acc = jnp.zeros((LANES,), jnp.float32)
        ninf = jnp.full((LANES,), NEG, dtype=jnp.float32)
        for r in range(RPW):
            row = row_base + wid * RPW + r
            pltpu.sync_copy(logits_hbm.at[row], buf)

            def body(j, carry):
                new = []
                base = j * (LANES * NTRIO)
                for t in range(NTRIO):
                    a, b, c = carry[3 * t:3 * t + 3]
                    x = buf[pl.ds(base + t * LANES, LANES)]
                    a2 = jnp.maximum(a, x)
                    tt = jnp.minimum(a, x)
                    b2 = jnp.maximum(b, tt)
                    tt2 = jnp.minimum(b, tt)
                    c2 = jnp.maximum(c, tt2)
                    new += [a2, b2, c2]
                return tuple(new)

            trios = lax.fori_loop(0, NVREG, body, (ninf,) * (3 * NTRIO))
            a, b, c = trios[0:3]
            for t in range(1, NTRIO):
                a, b, c = _merge_sorted3(a, b, c, *trios[3 * t:3 * t + 3])
            for off in (8, 4, 2, 1):
                av[...] = a
                bv[...] = b
                cv[...] = c
                ix = jnp.bitwise_xor(iota, off)
                a_s = plsc.load_gather(av, [ix])
                b_s = plsc.load_gather(bv, [ix])
                c_s = plsc.load_gather(cv, [ix])
                a, b, c = _merge_sorted3(a, b, c, a_s, b_s, c_s)
            zy = plsc.load_gather(buf, [yvec])
            z_other = jnp.where((zy == a) & (b < a), b, a)
            scale = a - c + jnp.float32(1e-12)
            loss_vec = -(zy - z_other) / scale
            out_acc = jnp.where(iota == r, loss_vec, out_acc)
        outv[...] = out_acc
        pltpu.sync_copy(outv, out_hbm.at[wid])

    return dlr_loss_sc


_sc_call_lo = _make_sc_call(0)
_sc_call_hi = _make_sc_call(B // 2)


def kernel(logits, y_true):
    y32 = y_true.astype(jnp.int32).reshape(2 * NSUB, RPW)
    ypad = jnp.zeros((2 * NSUB, LANES), jnp.int32).at[:, :RPW].set(y32)
    out_lo = _sc_call_lo(logits, ypad[:NSUB])
    out_hi = _sc_call_hi(logits, ypad[NSUB:])
    out = jnp.concatenate([out_lo[:, :RPW], out_hi[:, :RPW]], axis=0)
    return out.reshape(B)


# speedup vs baseline: 1.3394x; 1.2914x over previous
import functools

import jax
import jax.numpy as jnp
from jax import lax
from jax.experimental import pallas as pl
from jax.experimental.pallas import tpu as pltpu
from jax.experimental.pallas import tpu_sc as plsc

B = 128
V = 100000
BT = 96
NSC = B - BT
NW = 32
RPW = NSC // NW
LANES = 16
NTRIO = 5
NVREG = V // (LANES * NTRIO)
WCH = 1000
NCCH = V // WCH
NEG = float("-inf")


def _merge_sorted3(a, b, c, a2, b2, c2):
    x1 = jnp.maximum(a, a2)
    y1 = jnp.minimum(a, a2)
    x2 = jnp.maximum(b, b2)
    y2 = jnp.minimum(b, b2)
    x3 = jnp.maximum(c, c2)
    m2 = jnp.maximum(y1, x2)
    m3 = jnp.maximum(jnp.minimum(y1, x2), jnp.maximum(y2, x3))
    return x1, m2, m3


def _make_sc_call():
    mesh = plsc.VectorSubcoreMesh(core_axis_name="c", subcore_axis_name="s")

    @functools.partial(
        pl.kernel,
        mesh=mesh,
        compiler_params=pltpu.CompilerParams(needs_layout_passes=False),
        out_type=jax.ShapeDtypeStruct((NW, LANES), jnp.float32),
        scratch_types=[
            pltpu.VMEM((V,), jnp.float32),
            pltpu.VMEM((LANES,), jnp.int32),
            pltpu.VMEM((LANES,), jnp.float32),
            pltpu.VMEM((LANES,), jnp.float32),
            pltpu.VMEM((LANES,), jnp.float32),
            pltpu.VMEM((LANES,), jnp.float32),
        ],
    )
    def dlr_loss_sc(logits_hbm, ypad_hbm, out_hbm, buf, yv, av, bv, cv, outv):
        wid = lax.axis_index("s") * 2 + lax.axis_index("c")
        pltpu.sync_copy(ypad_hbm.at[wid], yv)
        yvec = yv[...]
        iota = lax.iota(jnp.int32, LANES)
        out_acc = jnp.zeros((LANES,), jnp.float32)
        ninf = jnp.full((LANES,), NEG, dtype=jnp.float32)
        for r in range(RPW):
            row = BT + wid * RPW + r
            pltpu.sync_copy(logits_hbm.at[row], buf)

            def body(j, carry):
                new = []
                base = j * (LANES * NTRIO)
                for t in range(NTRIO):
                    a, b, c = carry[3 * t:3 * t + 3]
                    x = buf[pl.ds(base + t * LANES, LANES)]
                    a2 = jnp.maximum(a, x)
                    tt = jnp.minimum(a, x)
                    b2 = jnp.maximum(b, tt)
                    tt2 = jnp.minimum(b, tt)
                    c2 = jnp.maximum(c, tt2)
                    new += [a2, b2, c2]
                return tuple(new)

            trios = lax.fori_loop(0, NVREG, body, (ninf,) * (3 * NTRIO))
            a, b, c = trios[0:3]
            for t in range(1, NTRIO):
                a, b, c = _merge_sorted3(a, b, c, *trios[3 * t:3 * t + 3])
            for off in (8, 4, 2, 1):
                av[...] = a
                bv[...] = b
                cv[...] = c
                ix = jnp.bitwise_xor(iota, off)
                a_s = plsc.load_gather(av, [ix])
                b_s = plsc.load_gather(bv, [ix])
                c_s = plsc.load_gather(cv, [ix])
                a, b, c = _merge_sorted3(a, b, c, a_s, b_s, c_s)
            zy = plsc.load_gather(buf, [yvec])
            z_other = jnp.where((zy == a) & (b < a), b, a)
            scale = a - c + jnp.float32(1e-12)
            loss_vec = -(zy - z_other) / scale
            out_acc = jnp.where(iota == r, loss_vec, out_acc)
        outv[...] = out_acc
        pltpu.sync_copy(outv, out_hbm.at[wid])

    return dlr_loss_sc


def _tc_body(x_ref, y_ref, o_ref):
    ycol = y_ref[:, 0:1]
    x = x_ref[...]
    ci = lax.broadcasted_iota(jnp.int32, (8, V), 1)
    ninf = jnp.float32(NEG)
    zy = jnp.max(jnp.where(ci == ycol, x, ninf), axis=1, keepdims=True)
    v1 = jnp.max(x, axis=1, keepdims=True)
    i1 = jnp.min(jnp.where(x == v1, ci, V), axis=1, keepdims=True)
    x2 = jnp.where(ci == i1, ninf, x)
    v2 = jnp.max(x2, axis=1, keepdims=True)
    i2 = jnp.min(jnp.where(x2 == v2, ci, V), axis=1, keepdims=True)
    x3 = jnp.where(ci == i2, ninf, x2)
    v3 = jnp.max(x3, axis=1, keepdims=True)
    z_other = jnp.where((zy == v1) & (v2 < v1), v2, v1)
    loss = -(zy - z_other) / (v1 - v3 + jnp.float32(1e-12))
    o_ref[...] = jnp.broadcast_to(loss, (8, 8))


def _make_tc_call():
    return pl.pallas_call(
        _tc_body,
        grid=(BT // 8,),
        in_specs=[
            pl.BlockSpec((8, V), lambda i: (i, 0)),
            pl.BlockSpec((8, 8), lambda i: (i, 0)),
        ],
        out_specs=pl.BlockSpec((8, 8), lambda i: (i, 0)),
        out_shape=jax.ShapeDtypeStruct((BT, 8), jnp.float32),
    )


_sc_call = _make_sc_call()
_tc_call = _make_tc_call()


def kernel(logits, y_true):
    y32 = y_true.astype(jnp.int32)
    ybc = jnp.broadcast_to(y32[:BT, None], (BT, 8))
    loss_tc = _tc_call(logits, ybc)[:, 0]
    ypad = jnp.zeros((NW, LANES), jnp.int32).at[:, :RPW].set(
        y32[BT:].reshape(NW, RPW))
    out_sc = _sc_call(logits, ypad)
    loss_sc = out_sc[:, :RPW].reshape(NSC)
    return jnp.concatenate([loss_tc, loss_sc])
